# Initial kernel scaffold; baseline (speedup 1.0000x reference)
import jax
import jax.numpy as jnp
from jax.experimental import pallas as pl


def kernel(hidden_states, gru_w_ih, gru_w_hh, expr_W, ctx_W, ctx_b, temp, expert_wg, expert_wu, expert_wd, shared_wg, shared_wu, shared_wd):
    raise NotImplementedError("write your pallas kernel here")


# speedup vs baseline: 10.1512x; 10.1512x over previous
import functools

import jax
import jax.numpy as jnp
from jax.experimental import pallas as pl
from jax.experimental.pallas import tpu as pltpu

F32 = jnp.float32
HI = jax.lax.Precision.HIGHEST


def _pre_kernel(x_ref, wihT_ref, exprWT_ref, ctxWT_ref, ctxb_ref,
                xw_ref, expr_ref, hn0_ref, xsum_ref, *, nT, S):
    t = pl.program_id(0)
    x = x_ref[...]
    xw_ref[...] = jnp.dot(x, wihT_ref[...], preferred_element_type=F32)
    expr_ref[...] = jnp.dot(x, exprWT_ref[...], preferred_element_type=F32)
    part = jnp.sum(x, axis=0, keepdims=True)

    @pl.when(t == 0)
    def _():
        xsum_ref[...] = part

    @pl.when(t != 0)
    def _():
        xsum_ref[...] += part

    @pl.when(t == nT - 1)
    def _():
        mean = xsum_ref[...] * (1.0 / S)
        hn0_ref[...] = jnp.dot(mean, ctxWT_ref[...],
                               preferred_element_type=F32) + ctxb_ref[...]


def _gru_kernel(xw_ref, whhT_ref, hn0_ref, rout_ref, h_scr, *, CG, GH):
    c = pl.program_id(0)

    @pl.when(c == 0)
    def _():
        h_scr[...] = hn0_ref[...]

    whhT = whhT_ref[...]
    h0 = h_scr[...]

    def step(i, h):
        xw = xw_ref[pl.ds(i, 1), :]
        hh = jnp.dot(h, whhT, preferred_element_type=F32)
        r = jax.nn.sigmoid(xw[:, :GH] + hh[:, :GH])
        z = jax.nn.sigmoid(xw[:, GH:2 * GH] + hh[:, GH:2 * GH])
        n = jnp.tanh(xw[:, 2 * GH:] + r * hh[:, 2 * GH:])
        h_new = (1.0 - z) * n + z * h
        rout_ref[pl.ds(i, 1), :] = h_new
        return h_new

    h_fin = jax.lax.fori_loop(0, CG, step, h0)
    h_scr[...] = h_fin


def _stats_kernel(rout_ref, expr_ref, cs_ref, pen_ref, *, E, RD, GH):
    t = pl.program_id(0)
    r = rout_ref[...]
    ex = expr_ref[...]
    seg = (jax.lax.broadcasted_iota(jnp.int32, (GH, E), 0) // RD)
    M = (seg == jax.lax.broadcasted_iota(jnp.int32, (GH, E), 1)).astype(F32)

    rn2 = jnp.dot(r * r, M, preferred_element_type=F32, precision=HI)
    inv = 1.0 / jnp.maximum(jnp.sqrt(rn2), 1e-12)
    inv_big = jnp.dot(inv, M.T, preferred_element_type=F32, precision=HI)
    normed = r * inv_big

    nn2 = jnp.dot(normed * normed, M, preferred_element_type=F32, precision=HI)
    en2 = jnp.dot(ex * ex, M, preferred_element_type=F32, precision=HI)
    num = jnp.dot(ex * normed, M, preferred_element_type=F32, precision=HI)
    den = jnp.maximum(jnp.sqrt(en2) * jnp.sqrt(nn2), 1e-8)
    cs_ref[...] = 1.0 - num / den

    acc = jnp.zeros((r.shape[0],), dtype=F32)
    for i in range(E):
        si = normed[:, i * RD:(i + 1) * RD]
        rep = jnp.concatenate([si] * E, axis=1)
        grow = jnp.dot(normed * rep, M, preferred_element_type=F32, precision=HI)
        onei = (jax.lax.broadcasted_iota(jnp.int32, grow.shape, 1) == i)
        diff = grow - onei.astype(F32)
        rown2 = jnp.sum(diff * diff, axis=1)
        rown = jnp.maximum(jnp.sqrt(rown2), 1e-12)
        acc = acc + rown2 / (rown * rown)
    part = jnp.sum(acc).reshape(1, 1)

    @pl.when(t == 0)
    def _():
        pen_ref[...] = part

    @pl.when(t != 0)
    def _():
        pen_ref[...] += part


def _topk_kernel(cs_ref, pen_ref, temp_ref, comb_ref, *, E, T):
    pen = pen_ref[0, 0] * (1.0 / T)
    tmp = temp_ref[0, 0]
    s = cs_ref[...] * (1.0 + pen)
    idx = jax.lax.broadcasted_iota(jnp.int32, s.shape, 1)
    v1 = jnp.max(s, axis=1, keepdims=True)
    i1 = jnp.min(jnp.where(s == v1, idx, E), axis=1, keepdims=True)
    masked = jnp.where(idx == i1, -jnp.inf, s)
    v2 = jnp.max(masked, axis=1, keepdims=True)
    i2 = jnp.min(jnp.where(masked == v2, idx, E), axis=1, keepdims=True)
    e2 = jnp.exp((v2 - v1) / tmp)
    w1 = 1.0 / (1.0 + e2)
    w2 = e2 / (1.0 + e2)
    comb_ref[...] = (jnp.where(idx == i1, w1, 0.0)
                     + jnp.where(idx == i2, w2, 0.0))


def _experts_kernel(x_ref, wg_ref, wu_ref, wd_ref, comb_ref, out_ref, *, BT):
    e = pl.program_id(0)
    t = pl.program_id(1)
    x = x_ref[...]
    g = jax.lax.dot_general(x, wg_ref[0], (((1,), (1,)), ((), ())),
                            preferred_element_type=F32)
    u = jax.lax.dot_general(x, wu_ref[0], (((1,), (1,)), ((), ())),
                            preferred_element_type=F32)
    y = g * jax.nn.sigmoid(g) * u
    o = jax.lax.dot_general(y, wd_ref[0], (((1,), (1,)), ((), ())),
                            preferred_element_type=F32)

    @pl.when(e == 0)
    def _():
        out_ref[pl.ds(t * BT, BT), :] = o

    @pl.when(e != 0)
    def _():
        comb = comb_ref[...]
        lane = jax.lax.broadcasted_iota(jnp.int32, comb.shape, 1)
        w = jnp.sum(jnp.where(lane == e - 1, comb, 0.0), axis=1, keepdims=True)
        out_ref[pl.ds(t * BT, BT), :] += o * w


def kernel(hidden_states, gru_w_ih, gru_w_hh, expr_W, ctx_W, ctx_b, temp,
           expert_wg, expert_wu, expert_wd, shared_wg, shared_wu, shared_wd):
    b, S, H = hidden_states.shape
    GH = gru_w_hh.shape[1]
    threeGH = gru_w_ih.shape[0]
    E = expert_wg.shape[0]
    RD = GH // E
    I = expert_wg.shape[1]
    T = b * S

    x = hidden_states.reshape(T, H)
    wihT = gru_w_ih.T
    whhT = gru_w_hh.T
    exprWT = expr_W.T
    ctxWT = ctx_W.T
    ctxb2 = ctx_b.reshape(1, GH)
    temp2 = temp.reshape(1, 1)

    BT1 = min(256, T)
    nT1 = T // BT1
    xw, expr, hn0, _xsum = pl.pallas_call(
        functools.partial(_pre_kernel, nT=nT1, S=T),
        grid=(nT1,),
        in_specs=[
            pl.BlockSpec((BT1, H), lambda t: (t, 0)),
            pl.BlockSpec((H, threeGH), lambda t: (0, 0)),
            pl.BlockSpec((H, GH), lambda t: (0, 0)),
            pl.BlockSpec((H, GH), lambda t: (0, 0)),
            pl.BlockSpec((1, GH), lambda t: (0, 0)),
        ],
        out_specs=[
            pl.BlockSpec((BT1, threeGH), lambda t: (t, 0)),
            pl.BlockSpec((BT1, GH), lambda t: (t, 0)),
            pl.BlockSpec((1, GH), lambda t: (0, 0)),
            pl.BlockSpec((1, H), lambda t: (0, 0)),
        ],
        out_shape=[
            jax.ShapeDtypeStruct((T, threeGH), F32),
            jax.ShapeDtypeStruct((T, GH), F32),
            jax.ShapeDtypeStruct((1, GH), F32),
            jax.ShapeDtypeStruct((1, H), F32),
        ],
    )(x, wihT, exprWT, ctxWT, ctxb2)

    CG = 64
    nC = T // CG
    routing = pl.pallas_call(
        functools.partial(_gru_kernel, CG=CG, GH=GH),
        grid=(nC,),
        in_specs=[
            pl.BlockSpec((CG, threeGH), lambda c: (c, 0)),
            pl.BlockSpec((GH, threeGH), lambda c: (0, 0)),
            pl.BlockSpec((1, GH), lambda c: (0, 0)),
        ],
        out_specs=pl.BlockSpec((CG, GH), lambda c: (c, 0)),
        out_shape=jax.ShapeDtypeStruct((T, GH), F32),
        scratch_shapes=[pltpu.VMEM((1, GH), F32)],
    )(xw, whhT, hn0)

    BT3 = min(256, T)
    nT3 = T // BT3
    cs, pen = pl.pallas_call(
        functools.partial(_stats_kernel, E=E, RD=RD, GH=GH),
        grid=(nT3,),
        in_specs=[
            pl.BlockSpec((BT3, GH), lambda t: (t, 0)),
            pl.BlockSpec((BT3, GH), lambda t: (t, 0)),
        ],
        out_specs=[
            pl.BlockSpec((BT3, E), lambda t: (t, 0)),
            pl.BlockSpec((1, 1), lambda t: (0, 0)),
        ],
        out_shape=[
            jax.ShapeDtypeStruct((T, E), F32),
            jax.ShapeDtypeStruct((1, 1), F32),
        ],
    )(routing, expr)

    combine = pl.pallas_call(
        functools.partial(_topk_kernel, E=E, T=T),
        grid=(1,),
        in_specs=[
            pl.BlockSpec((T, E), lambda i: (0, 0)),
            pl.BlockSpec((1, 1), lambda i: (0, 0)),
            pl.BlockSpec((1, 1), lambda i: (0, 0)),
        ],
        out_specs=pl.BlockSpec((T, E), lambda i: (0, 0)),
        out_shape=jax.ShapeDtypeStruct((T, E), F32),
    )(cs, pen, temp2)

    wgS = jnp.concatenate([shared_wg[None], expert_wg], axis=0)
    wuS = jnp.concatenate([shared_wu[None], expert_wu], axis=0)
    wdS = jnp.concatenate([shared_wd[None], expert_wd], axis=0)
    BT4 = min(512, T)
    nT4 = T // BT4
    out = pl.pallas_call(
        functools.partial(_experts_kernel, BT=BT4),
        grid=(E + 1, nT4),
        in_specs=[
            pl.BlockSpec((BT4, H), lambda e, t: (t, 0)),
            pl.BlockSpec((1, I, H), lambda e, t: (e, 0, 0)),
            pl.BlockSpec((1, I, H), lambda e, t: (e, 0, 0)),
            pl.BlockSpec((1, H, I), lambda e, t: (e, 0, 0)),
            pl.BlockSpec((BT4, E), lambda e, t: (t, 0)),
        ],
        out_specs=pl.BlockSpec((T, H), lambda e, t: (0, 0)),
        out_shape=jax.ShapeDtypeStruct((T, H), F32),
    )(x, wgS, wuS, wdS, combine)

    return out.reshape(b, S, H)


# speedup vs baseline: 10.5720x; 1.0415x over previous
import functools

import jax
import jax.numpy as jnp
from jax.experimental import pallas as pl
from jax.experimental.pallas import tpu as pltpu

F32 = jnp.float32
HI = jax.lax.Precision.HIGHEST


def _pre_kernel(x_ref, wihT_ref, exprWT_ref, ctxWT_ref, ctxb_ref,
                xw_ref, expr_ref, hn0_ref, xsum_ref, *, nT, S):
    t = pl.program_id(0)
    x = x_ref[...]
    xw_ref[...] = jnp.dot(x, wihT_ref[...], preferred_element_type=F32)
    expr_ref[...] = jnp.dot(x, exprWT_ref[...], preferred_element_type=F32)
    part = jnp.sum(x, axis=0, keepdims=True)

    @pl.when(t == 0)
    def _():
        xsum_ref[...] = part

    @pl.when(t != 0)
    def _():
        xsum_ref[...] += part

    @pl.when(t == nT - 1)
    def _():
        mean = xsum_ref[...] * (1.0 / S)
        hn0_ref[...] = jnp.dot(mean, ctxWT_ref[...],
                               preferred_element_type=F32) + ctxb_ref[...]


def _gru_kernel(xw_ref, whhT_ref, hn0_ref, rout_ref, h_scr, *, CG, GH):
    c = pl.program_id(0)

    @pl.when(c == 0)
    def _():
        h_scr[...] = hn0_ref[...]

    whhT = whhT_ref[...]
    h0 = h_scr[...]

    def step(i, h):
        xw = xw_ref[pl.ds(i, 1), :]
        hh_r = jnp.dot(h, whhT[:, :GH], preferred_element_type=F32)
        hh_z = jnp.dot(h, whhT[:, GH:2 * GH], preferred_element_type=F32)
        hh_n = jnp.dot(h, whhT[:, 2 * GH:], preferred_element_type=F32)
        r = jax.nn.sigmoid(xw[:, :GH] + hh_r)
        z = jax.nn.sigmoid(xw[:, GH:2 * GH] + hh_z)
        n = jnp.tanh(xw[:, 2 * GH:] + r * hh_n)
        h_new = (1.0 - z) * n + z * h
        rout_ref[pl.ds(i, 1), :] = h_new
        return h_new

    h_fin = jax.lax.fori_loop(0, CG, step, h0)
    h_scr[...] = h_fin


def _stats_kernel(rout_ref, expr_ref, cs_ref, pen_ref, *, E, RD, GH):
    t = pl.program_id(0)
    r = rout_ref[...]
    ex = expr_ref[...]
    seg = (jax.lax.broadcasted_iota(jnp.int32, (GH, E), 0) // RD)
    M = (seg == jax.lax.broadcasted_iota(jnp.int32, (GH, E), 1)).astype(F32)

    rn2 = jnp.dot(r * r, M, preferred_element_type=F32, precision=HI)
    inv = 1.0 / jnp.maximum(jnp.sqrt(rn2), 1e-12)
    inv_big = jnp.dot(inv, M.T, preferred_element_type=F32, precision=HI)
    normed = r * inv_big

    nn2 = jnp.dot(normed * normed, M, preferred_element_type=F32, precision=HI)
    en2 = jnp.dot(ex * ex, M, preferred_element_type=F32, precision=HI)
    num = jnp.dot(ex * normed, M, preferred_element_type=F32, precision=HI)
    den = jnp.maximum(jnp.sqrt(en2) * jnp.sqrt(nn2), 1e-8)
    cs_ref[...] = 1.0 - num / den

    acc = jnp.zeros((r.shape[0],), dtype=F32)
    for i in range(E):
        si = normed[:, i * RD:(i + 1) * RD]
        rep = jnp.concatenate([si] * E, axis=1)
        grow = jnp.dot(normed * rep, M, preferred_element_type=F32, precision=HI)
        onei = (jax.lax.broadcasted_iota(jnp.int32, grow.shape, 1) == i)
        diff = grow - onei.astype(F32)
        rown2 = jnp.sum(diff * diff, axis=1)
        rown = jnp.maximum(jnp.sqrt(rown2), 1e-12)
        acc = acc + rown2 / (rown * rown)
    part = jnp.sum(acc).reshape(1, 1)

    @pl.when(t == 0)
    def _():
        pen_ref[...] = part

    @pl.when(t != 0)
    def _():
        pen_ref[...] += part


def _topk_kernel(cs_ref, pen_ref, temp_ref, comb_ref, *, E, T):
    pen = pen_ref[0, 0] * (1.0 / T)
    tmp = temp_ref[0, 0]
    s = cs_ref[...] * (1.0 + pen)
    idx = jax.lax.broadcasted_iota(jnp.int32, s.shape, 1)
    v1 = jnp.max(s, axis=1, keepdims=True)
    i1 = jnp.min(jnp.where(s == v1, idx, E), axis=1, keepdims=True)
    masked = jnp.where(idx == i1, -jnp.inf, s)
    v2 = jnp.max(masked, axis=1, keepdims=True)
    i2 = jnp.min(jnp.where(masked == v2, idx, E), axis=1, keepdims=True)
    e2 = jnp.exp((v2 - v1) / tmp)
    w1 = 1.0 / (1.0 + e2)
    w2 = e2 / (1.0 + e2)
    comb_ref[...] = (jnp.where(idx == i1, w1, 0.0)
                     + jnp.where(idx == i2, w2, 0.0))


def _shared_kernel(x_ref, wg_ref, wu_ref, wd_ref, out_ref):
    x = x_ref[...]
    g = jax.lax.dot_general(x, wg_ref[...], (((1,), (1,)), ((), ())),
                            preferred_element_type=F32)
    u = jax.lax.dot_general(x, wu_ref[...], (((1,), (1,)), ((), ())),
                            preferred_element_type=F32)
    y = g * jax.nn.sigmoid(g) * u
    out_ref[...] = jax.lax.dot_general(y, wd_ref[...], (((1,), (1,)), ((), ())),
                                       preferred_element_type=F32)


def _experts_kernel(x_ref, wg_ref, wu_ref, wd_ref, comb_ref, base_ref, out_ref,
                    *, BT):
    e = pl.program_id(0)
    t = pl.program_id(1)
    x = x_ref[...]
    g = jax.lax.dot_general(x, wg_ref[0], (((1,), (1,)), ((), ())),
                            preferred_element_type=F32)
    u = jax.lax.dot_general(x, wu_ref[0], (((1,), (1,)), ((), ())),
                            preferred_element_type=F32)
    y = g * jax.nn.sigmoid(g) * u
    o = jax.lax.dot_general(y, wd_ref[0], (((1,), (1,)), ((), ())),
                            preferred_element_type=F32)
    comb = comb_ref[...]
    lane = jax.lax.broadcasted_iota(jnp.int32, comb.shape, 1)
    w = jnp.sum(jnp.where(lane == e, comb, 0.0), axis=1, keepdims=True)

    @pl.when(e == 0)
    def _():
        out_ref[pl.ds(t * BT, BT), :] = base_ref[pl.ds(t * BT, BT), :] + o * w

    @pl.when(e != 0)
    def _():
        out_ref[pl.ds(t * BT, BT), :] += o * w


def kernel(hidden_states, gru_w_ih, gru_w_hh, expr_W, ctx_W, ctx_b, temp,
           expert_wg, expert_wu, expert_wd, shared_wg, shared_wu, shared_wd):
    b, S, H = hidden_states.shape
    GH = gru_w_hh.shape[1]
    threeGH = gru_w_ih.shape[0]
    E = expert_wg.shape[0]
    RD = GH // E
    I = expert_wg.shape[1]
    T = b * S

    x = hidden_states.reshape(T, H)
    wihT = gru_w_ih.T
    whhT = gru_w_hh.T
    exprWT = expr_W.T
    ctxWT = ctx_W.T
    ctxb2 = ctx_b.reshape(1, GH)
    temp2 = temp.reshape(1, 1)

    BT1 = min(256, T)
    nT1 = T // BT1
    xw, expr, hn0, _xsum = pl.pallas_call(
        functools.partial(_pre_kernel, nT=nT1, S=T),
        grid=(nT1,),
        in_specs=[
            pl.BlockSpec((BT1, H), lambda t: (t, 0)),
            pl.BlockSpec((H, threeGH), lambda t: (0, 0)),
            pl.BlockSpec((H, GH), lambda t: (0, 0)),
            pl.BlockSpec((H, GH), lambda t: (0, 0)),
            pl.BlockSpec((1, GH), lambda t: (0, 0)),
        ],
        out_specs=[
            pl.BlockSpec((BT1, threeGH), lambda t: (t, 0)),
            pl.BlockSpec((BT1, GH), lambda t: (t, 0)),
            pl.BlockSpec((1, GH), lambda t: (0, 0)),
            pl.BlockSpec((1, H), lambda t: (0, 0)),
        ],
        out_shape=[
            jax.ShapeDtypeStruct((T, threeGH), F32),
            jax.ShapeDtypeStruct((T, GH), F32),
            jax.ShapeDtypeStruct((1, GH), F32),
            jax.ShapeDtypeStruct((1, H), F32),
        ],
    )(x, wihT, exprWT, ctxWT, ctxb2)

    CG = 64
    nC = T // CG
    routing = pl.pallas_call(
        functools.partial(_gru_kernel, CG=CG, GH=GH),
        grid=(nC,),
        in_specs=[
            pl.BlockSpec((CG, threeGH), lambda c: (c, 0)),
            pl.BlockSpec((GH, threeGH), lambda c: (0, 0)),
            pl.BlockSpec((1, GH), lambda c: (0, 0)),
        ],
        out_specs=pl.BlockSpec((CG, GH), lambda c: (c, 0)),
        out_shape=jax.ShapeDtypeStruct((T, GH), F32),
        scratch_shapes=[pltpu.VMEM((1, GH), F32)],
    )(xw, whhT, hn0)

    BT3 = min(256, T)
    nT3 = T // BT3
    cs, pen = pl.pallas_call(
        functools.partial(_stats_kernel, E=E, RD=RD, GH=GH),
        grid=(nT3,),
        in_specs=[
            pl.BlockSpec((BT3, GH), lambda t: (t, 0)),
            pl.BlockSpec((BT3, GH), lambda t: (t, 0)),
        ],
        out_specs=[
            pl.BlockSpec((BT3, E), lambda t: (t, 0)),
            pl.BlockSpec((1, 1), lambda t: (0, 0)),
        ],
        out_shape=[
            jax.ShapeDtypeStruct((T, E), F32),
            jax.ShapeDtypeStruct((1, 1), F32),
        ],
    )(routing, expr)

    combine = pl.pallas_call(
        functools.partial(_topk_kernel, E=E, T=T),
        grid=(1,),
        in_specs=[
            pl.BlockSpec((T, E), lambda i: (0, 0)),
            pl.BlockSpec((1, 1), lambda i: (0, 0)),
            pl.BlockSpec((1, 1), lambda i: (0, 0)),
        ],
        out_specs=pl.BlockSpec((T, E), lambda i: (0, 0)),
        out_shape=jax.ShapeDtypeStruct((T, E), F32),
    )(cs, pen, temp2)

    BT4 = min(512, T)
    nT4 = T // BT4
    shared_out = pl.pallas_call(
        _shared_kernel,
        grid=(nT4,),
        in_specs=[
            pl.BlockSpec((BT4, H), lambda t: (t, 0)),
            pl.BlockSpec((I, H), lambda t: (0, 0)),
            pl.BlockSpec((I, H), lambda t: (0, 0)),
            pl.BlockSpec((H, I), lambda t: (0, 0)),
        ],
        out_specs=pl.BlockSpec((BT4, H), lambda t: (t, 0)),
        out_shape=jax.ShapeDtypeStruct((T, H), F32),
    )(x, shared_wg, shared_wu, shared_wd)

    out = pl.pallas_call(
        functools.partial(_experts_kernel, BT=BT4),
        grid=(E, nT4),
        in_specs=[
            pl.BlockSpec((BT4, H), lambda e, t: (t, 0)),
            pl.BlockSpec((1, I, H), lambda e, t: (e, 0, 0)),
            pl.BlockSpec((1, I, H), lambda e, t: (e, 0, 0)),
            pl.BlockSpec((1, H, I), lambda e, t: (e, 0, 0)),
            pl.BlockSpec((BT4, E), lambda e, t: (t, 0)),
            pl.BlockSpec((T, H), lambda e, t: (0, 0)),
        ],
        out_specs=pl.BlockSpec((T, H), lambda e, t: (0, 0)),
        out_shape=jax.ShapeDtypeStruct((T, H), F32),
    )(x, expert_wg, expert_wu, expert_wd, combine, shared_out)

    return out.reshape(b, S, H)


# speedup vs baseline: 10.6318x; 1.0057x over previous
import functools

import jax
import jax.numpy as jnp
from jax.experimental import pallas as pl
from jax.experimental.pallas import tpu as pltpu

F32 = jnp.float32
HI = jax.lax.Precision.HIGHEST


def _pre_kernel(x_ref, wihT_ref, exprWT_ref, ctxWT_ref, ctxb_ref,
                xw_ref, expr_ref, hn0_ref, xsum_ref, *, nT, S):
    t = pl.program_id(0)
    x = x_ref[...]
    xw_ref[...] = jnp.dot(x, wihT_ref[...], preferred_element_type=F32)
    expr_ref[...] = jnp.dot(x, exprWT_ref[...], preferred_element_type=F32)
    part = jnp.sum(x, axis=0, keepdims=True)

    @pl.when(t == 0)
    def _():
        xsum_ref[...] = part

    @pl.when(t != 0)
    def _():
        xsum_ref[...] += part

    @pl.when(t == nT - 1)
    def _():
        mean = xsum_ref[...] * (1.0 / S)
        hn0_ref[...] = jnp.dot(mean, ctxWT_ref[...],
                               preferred_element_type=F32) + ctxb_ref[...]


def _gru_kernel(xw_ref, whhT_ref, hn0_ref, rout_ref, h_scr, *, CG, GH):
    c = pl.program_id(0)

    @pl.when(c == 0)
    def _():
        h_scr[...] = hn0_ref[...]

    whhT = whhT_ref[...]
    h0 = h_scr[...]

    def step(i, h):
        xw = xw_ref[pl.ds(i, 1), :]
        hh_r = jnp.dot(h, whhT[:, :GH], preferred_element_type=F32)
        hh_z = jnp.dot(h, whhT[:, GH:2 * GH], preferred_element_type=F32)
        hh_n = jnp.dot(h, whhT[:, 2 * GH:], preferred_element_type=F32)
        r = jax.nn.sigmoid(xw[:, :GH] + hh_r)
        z = jax.nn.sigmoid(xw[:, GH:2 * GH] + hh_z)
        n = jnp.tanh(xw[:, 2 * GH:] + r * hh_n)
        h_new = (1.0 - z) * n + z * h
        rout_ref[pl.ds(i, 1), :] = h_new
        return h_new

    h_fin = jax.lax.fori_loop(0, CG, step, h0)
    h_scr[...] = h_fin


def _stats_kernel(rout_ref, expr_ref, cs_ref, pen_ref, *, E, RD, GH):
    t = pl.program_id(0)
    r = rout_ref[...]
    ex = expr_ref[...]
    seg = (jax.lax.broadcasted_iota(jnp.int32, (GH, E), 0) // RD)
    M = (seg == jax.lax.broadcasted_iota(jnp.int32, (GH, E), 1)).astype(F32)

    rn2 = jnp.dot(r * r, M, preferred_element_type=F32, precision=HI)
    inv = 1.0 / jnp.maximum(jnp.sqrt(rn2), 1e-12)
    inv_big = jnp.dot(inv, M.T, preferred_element_type=F32, precision=HI)
    normed = r * inv_big

    nn2 = jnp.dot(normed * normed, M, preferred_element_type=F32, precision=HI)
    en2 = jnp.dot(ex * ex, M, preferred_element_type=F32, precision=HI)
    num = jnp.dot(ex * normed, M, preferred_element_type=F32, precision=HI)
    den = jnp.maximum(jnp.sqrt(en2) * jnp.sqrt(nn2), 1e-8)
    cs_ref[...] = 1.0 - num / den

    acc = jnp.zeros((r.shape[0],), dtype=F32)
    for i in range(E):
        si = normed[:, i * RD:(i + 1) * RD]
        rep = jnp.concatenate([si] * E, axis=1)
        grow = jnp.dot(normed * rep, M, preferred_element_type=F32, precision=HI)
        onei = (jax.lax.broadcasted_iota(jnp.int32, grow.shape, 1) == i)
        diff = grow - onei.astype(F32)
        rown2 = jnp.sum(diff * diff, axis=1)
        rown = jnp.maximum(jnp.sqrt(rown2), 1e-12)
        acc = acc + rown2 / (rown * rown)
    part = jnp.sum(acc).reshape(1, 1)

    @pl.when(t == 0)
    def _():
        pen_ref[...] = part

    @pl.when(t != 0)
    def _():
        pen_ref[...] += part


def _topk_kernel(cs_ref, pen_ref, temp_ref, comb_ref, *, E, T):
    pen = pen_ref[0, 0] * (1.0 / T)
    tmp = temp_ref[0, 0]
    s = cs_ref[...] * (1.0 + pen)
    idx = jax.lax.broadcasted_iota(jnp.int32, s.shape, 1)
    v1 = jnp.max(s, axis=1, keepdims=True)
    i1 = jnp.min(jnp.where(s == v1, idx, E), axis=1, keepdims=True)
    masked = jnp.where(idx == i1, -jnp.inf, s)
    v2 = jnp.max(masked, axis=1, keepdims=True)
    i2 = jnp.min(jnp.where(masked == v2, idx, E), axis=1, keepdims=True)
    e2 = jnp.exp((v2 - v1) / tmp)
    w1 = 1.0 / (1.0 + e2)
    w2 = e2 / (1.0 + e2)
    comb_ref[...] = (jnp.where(idx == i1, w1, 0.0)
                     + jnp.where(idx == i2, w2, 0.0))


def _shared_kernel(x_ref, wg_ref, wu_ref, wd_ref, out_ref):
    x = x_ref[...]
    g = jax.lax.dot_general(x, wg_ref[...], (((1,), (1,)), ((), ())),
                            preferred_element_type=F32)
    u = jax.lax.dot_general(x, wu_ref[...], (((1,), (1,)), ((), ())),
                            preferred_element_type=F32)
    y = g * jax.nn.sigmoid(g) * u
    out_ref[...] = jax.lax.dot_general(y, wd_ref[...], (((1,), (1,)), ((), ())),
                                       preferred_element_type=F32)


def _experts_kernel(x_ref, wg_ref, wu_ref, wd_ref, comb_ref, base_ref, out_ref,
                    *, BT):
    e = pl.program_id(0)
    t = pl.program_id(1)
    x = x_ref[...]
    g = jax.lax.dot_general(x, wg_ref[0], (((1,), (1,)), ((), ())),
                            preferred_element_type=F32)
    u = jax.lax.dot_general(x, wu_ref[0], (((1,), (1,)), ((), ())),
                            preferred_element_type=F32)
    y = g * jax.nn.sigmoid(g) * u
    o = jax.lax.dot_general(y, wd_ref[0], (((1,), (1,)), ((), ())),
                            preferred_element_type=F32)
    comb = comb_ref[...]
    lane = jax.lax.broadcasted_iota(jnp.int32, comb.shape, 1)
    w = jnp.sum(jnp.where(lane == e, comb, 0.0), axis=1, keepdims=True)

    @pl.when(e == 0)
    def _():
        out_ref[pl.ds(t * BT, BT), :] = base_ref[pl.ds(t * BT, BT), :] + o * w

    @pl.when(e != 0)
    def _():
        out_ref[pl.ds(t * BT, BT), :] += o * w


def kernel(hidden_states, gru_w_ih, gru_w_hh, expr_W, ctx_W, ctx_b, temp,
           expert_wg, expert_wu, expert_wd, shared_wg, shared_wu, shared_wd):
    b, S, H = hidden_states.shape
    GH = gru_w_hh.shape[1]
    threeGH = gru_w_ih.shape[0]
    E = expert_wg.shape[0]
    RD = GH // E
    I = expert_wg.shape[1]
    T = b * S

    x = hidden_states.reshape(T, H)
    wihT = gru_w_ih.T
    whhT = gru_w_hh.T
    exprWT = expr_W.T
    ctxWT = ctx_W.T
    ctxb2 = ctx_b.reshape(1, GH)
    temp2 = temp.reshape(1, 1)

    BT1 = min(256, T)
    nT1 = T // BT1
    xw, expr, hn0, _xsum = pl.pallas_call(
        functools.partial(_pre_kernel, nT=nT1, S=T),
        grid=(nT1,),
        in_specs=[
            pl.BlockSpec((BT1, H), lambda t: (t, 0)),
            pl.BlockSpec((H, threeGH), lambda t: (0, 0)),
            pl.BlockSpec((H, GH), lambda t: (0, 0)),
            pl.BlockSpec((H, GH), lambda t: (0, 0)),
            pl.BlockSpec((1, GH), lambda t: (0, 0)),
        ],
        out_specs=[
            pl.BlockSpec((BT1, threeGH), lambda t: (t, 0)),
            pl.BlockSpec((BT1, GH), lambda t: (t, 0)),
            pl.BlockSpec((1, GH), lambda t: (0, 0)),
            pl.BlockSpec((1, H), lambda t: (0, 0)),
        ],
        out_shape=[
            jax.ShapeDtypeStruct((T, threeGH), F32),
            jax.ShapeDtypeStruct((T, GH), F32),
            jax.ShapeDtypeStruct((1, GH), F32),
            jax.ShapeDtypeStruct((1, H), F32),
        ],
    )(x, wihT, exprWT, ctxWT, ctxb2)

    CG = 512
    nC = T // CG
    routing = pl.pallas_call(
        functools.partial(_gru_kernel, CG=CG, GH=GH),
        grid=(nC,),
        in_specs=[
            pl.BlockSpec((CG, threeGH), lambda c: (c, 0)),
            pl.BlockSpec((GH, threeGH), lambda c: (0, 0)),
            pl.BlockSpec((1, GH), lambda c: (0, 0)),
        ],
        out_specs=pl.BlockSpec((CG, GH), lambda c: (c, 0)),
        out_shape=jax.ShapeDtypeStruct((T, GH), F32),
        scratch_shapes=[pltpu.VMEM((1, GH), F32)],
    )(xw, whhT, hn0)

    BT3 = min(256, T)
    nT3 = T // BT3
    cs, pen = pl.pallas_call(
        functools.partial(_stats_kernel, E=E, RD=RD, GH=GH),
        grid=(nT3,),
        in_specs=[
            pl.BlockSpec((BT3, GH), lambda t: (t, 0)),
            pl.BlockSpec((BT3, GH), lambda t: (t, 0)),
        ],
        out_specs=[
            pl.BlockSpec((BT3, E), lambda t: (t, 0)),
            pl.BlockSpec((1, 1), lambda t: (0, 0)),
        ],
        out_shape=[
            jax.ShapeDtypeStruct((T, E), F32),
            jax.ShapeDtypeStruct((1, 1), F32),
        ],
    )(routing, expr)

    combine = pl.pallas_call(
        functools.partial(_topk_kernel, E=E, T=T),
        grid=(1,),
        in_specs=[
            pl.BlockSpec((T, E), lambda i: (0, 0)),
            pl.BlockSpec((1, 1), lambda i: (0, 0)),
            pl.BlockSpec((1, 1), lambda i: (0, 0)),
        ],
        out_specs=pl.BlockSpec((T, E), lambda i: (0, 0)),
        out_shape=jax.ShapeDtypeStruct((T, E), F32),
    )(cs, pen, temp2)

    BT4 = min(512, T)
    nT4 = T // BT4
    shared_out = pl.pallas_call(
        _shared_kernel,
        grid=(nT4,),
        in_specs=[
            pl.BlockSpec((BT4, H), lambda t: (t, 0)),
            pl.BlockSpec((I, H), lambda t: (0, 0)),
            pl.BlockSpec((I, H), lambda t: (0, 0)),
            pl.BlockSpec((H, I), lambda t: (0, 0)),
        ],
        out_specs=pl.BlockSpec((BT4, H), lambda t: (t, 0)),
        out_shape=jax.ShapeDtypeStruct((T, H), F32),
    )(x, shared_wg, shared_wu, shared_wd)

    out = pl.pallas_call(
        functools.partial(_experts_kernel, BT=BT4),
        grid=(E, nT4),
        in_specs=[
            pl.BlockSpec((BT4, H), lambda e, t: (t, 0)),
            pl.BlockSpec((1, I, H), lambda e, t: (e, 0, 0)),
            pl.BlockSpec((1, I, H), lambda e, t: (e, 0, 0)),
            pl.BlockSpec((1, H, I), lambda e, t: (e, 0, 0)),
            pl.BlockSpec((BT4, E), lambda e, t: (t, 0)),
            pl.BlockSpec((T, H), lambda e, t: (0, 0)),
        ],
        out_specs=pl.BlockSpec((T, H), lambda e, t: (0, 0)),
        out_shape=jax.ShapeDtypeStruct((T, H), F32),
    )(x, expert_wg, expert_wu, expert_wd, combine, shared_out)

    return out.reshape(b, S, H)


# speedup vs baseline: 10.8257x; 1.0182x over previous
import functools

import jax
import jax.numpy as jnp
from jax.experimental import pallas as pl
from jax.experimental.pallas import tpu as pltpu

F32 = jnp.float32
HI = jax.lax.Precision.HIGHEST


def _pre_kernel(x_ref, wihT_ref, exprWT_ref, ctxWT_ref, ctxb_ref,
                xw_ref, expr_ref, hn0_ref, xsum_ref, *, nT, S):
    t = pl.program_id(0)
    x = x_ref[...]
    xw_ref[...] = jnp.dot(x, wihT_ref[...], preferred_element_type=F32)
    expr_ref[...] = jnp.dot(x, exprWT_ref[...], preferred_element_type=F32)
    part = jnp.sum(x, axis=0, keepdims=True)

    @pl.when(t == 0)
    def _():
        xsum_ref[...] = part

    @pl.when(t != 0)
    def _():
        xsum_ref[...] += part

    @pl.when(t == nT - 1)
    def _():
        mean = xsum_ref[...] * (1.0 / S)
        hn0_ref[...] = jnp.dot(mean, ctxWT_ref[...],
                               preferred_element_type=F32) + ctxb_ref[...]


def _gru_kernel(xw_ref, whhT_ref, hn0_ref, rout_ref, h_scr, *, CG, GH):
    c = pl.program_id(0)

    @pl.when(c == 0)
    def _():
        h_scr[...] = hn0_ref[...]

    whhT = whhT_ref[...]
    h0 = h_scr[...]

    def step(i, h):
        xw = xw_ref[pl.ds(i, 1), :]
        hh_r = jnp.dot(h, whhT[:, :GH], preferred_element_type=F32)
        hh_z = jnp.dot(h, whhT[:, GH:2 * GH], preferred_element_type=F32)
        hh_n = jnp.dot(h, whhT[:, 2 * GH:], preferred_element_type=F32)
        r = jax.nn.sigmoid(xw[:, :GH] + hh_r)
        z = jax.nn.sigmoid(xw[:, GH:2 * GH] + hh_z)
        n = jnp.tanh(xw[:, 2 * GH:] + r * hh_n)
        h_new = (1.0 - z) * n + z * h
        rout_ref[pl.ds(i, 1), :] = h_new
        return h_new

    h_fin = jax.lax.fori_loop(0, CG, step, h0)
    h_scr[...] = h_fin


def _stats_kernel(rout_ref, expr_ref, cs_ref, pen_ref, *, E, RD, GH):
    t = pl.program_id(0)
    r = rout_ref[...]
    ex = expr_ref[...]
    seg = (jax.lax.broadcasted_iota(jnp.int32, (GH, E), 0) // RD)
    M = (seg == jax.lax.broadcasted_iota(jnp.int32, (GH, E), 1)).astype(F32)

    rn2 = jnp.dot(r * r, M, preferred_element_type=F32, precision=HI)
    inv = 1.0 / jnp.maximum(jnp.sqrt(rn2), 1e-12)
    inv_big = jnp.dot(inv, M.T, preferred_element_type=F32, precision=HI)
    normed = r * inv_big

    nn2 = jnp.dot(normed * normed, M, preferred_element_type=F32, precision=HI)
    en2 = jnp.dot(ex * ex, M, preferred_element_type=F32, precision=HI)
    num = jnp.dot(ex * normed, M, preferred_element_type=F32, precision=HI)
    den = jnp.maximum(jnp.sqrt(en2) * jnp.sqrt(nn2), 1e-8)
    cs_ref[...] = 1.0 - num / den

    normed_bf = normed.astype(jnp.bfloat16)
    M_bf = M.astype(jnp.bfloat16)
    acc = jnp.zeros((r.shape[0],), dtype=F32)
    for i in range(E):
        si = normed_bf[:, i * RD:(i + 1) * RD]
        rep = jnp.concatenate([si] * E, axis=1)
        grow = jnp.dot(normed_bf * rep, M_bf, preferred_element_type=F32)
        onei = (jax.lax.broadcasted_iota(jnp.int32, grow.shape, 1) == i)
        diff = grow - onei.astype(F32)
        rown2 = jnp.sum(diff * diff, axis=1)
        rown = jnp.maximum(jnp.sqrt(rown2), 1e-12)
        acc = acc + rown2 / (rown * rown)
    part = jnp.sum(acc).reshape(1, 1)

    @pl.when(t == 0)
    def _():
        pen_ref[...] = part

    @pl.when(t != 0)
    def _():
        pen_ref[...] += part


def _topk_kernel(cs_ref, pen_ref, temp_ref, comb_ref, *, E, T):
    pen = pen_ref[0, 0] * (1.0 / T)
    tmp = temp_ref[0, 0]
    s = cs_ref[...] * (1.0 + pen)
    idx = jax.lax.broadcasted_iota(jnp.int32, s.shape, 1)
    v1 = jnp.max(s, axis=1, keepdims=True)
    i1 = jnp.min(jnp.where(s == v1, idx, E), axis=1, keepdims=True)
    masked = jnp.where(idx == i1, -jnp.inf, s)
    v2 = jnp.max(masked, axis=1, keepdims=True)
    i2 = jnp.min(jnp.where(masked == v2, idx, E), axis=1, keepdims=True)
    e2 = jnp.exp((v2 - v1) / tmp)
    w1 = 1.0 / (1.0 + e2)
    w2 = e2 / (1.0 + e2)
    comb_ref[...] = (jnp.where(idx == i1, w1, 0.0)
                     + jnp.where(idx == i2, w2, 0.0))


def _shared_kernel(x_ref, wg_ref, wu_ref, wd_ref, out_ref):
    x = x_ref[...]
    g = jax.lax.dot_general(x, wg_ref[...], (((1,), (1,)), ((), ())),
                            preferred_element_type=F32)
    u = jax.lax.dot_general(x, wu_ref[...], (((1,), (1,)), ((), ())),
                            preferred_element_type=F32)
    y = g * jax.nn.sigmoid(g) * u
    out_ref[...] = jax.lax.dot_general(y, wd_ref[...], (((1,), (1,)), ((), ())),
                                       preferred_element_type=F32)


def _experts_kernel(x_ref, wg_ref, wu_ref, wd_ref, comb_ref, base_ref, out_ref,
                    *, BT):
    e = pl.program_id(0)
    t = pl.program_id(1)
    x = x_ref[...]
    g = jax.lax.dot_general(x, wg_ref[0], (((1,), (1,)), ((), ())),
                            preferred_element_type=F32)
    u = jax.lax.dot_general(x, wu_ref[0], (((1,), (1,)), ((), ())),
                            preferred_element_type=F32)
    y = g * jax.nn.sigmoid(g) * u
    o = jax.lax.dot_general(y, wd_ref[0], (((1,), (1,)), ((), ())),
                            preferred_element_type=F32)
    comb = comb_ref[...]
    lane = jax.lax.broadcasted_iota(jnp.int32, comb.shape, 1)
    w = jnp.sum(jnp.where(lane == e, comb, 0.0), axis=1, keepdims=True)

    @pl.when(e == 0)
    def _():
        out_ref[pl.ds(t * BT, BT), :] = base_ref[pl.ds(t * BT, BT), :] + o * w

    @pl.when(e != 0)
    def _():
        out_ref[pl.ds(t * BT, BT), :] += o * w


def kernel(hidden_states, gru_w_ih, gru_w_hh, expr_W, ctx_W, ctx_b, temp,
           expert_wg, expert_wu, expert_wd, shared_wg, shared_wu, shared_wd):
    b, S, H = hidden_states.shape
    GH = gru_w_hh.shape[1]
    threeGH = gru_w_ih.shape[0]
    E = expert_wg.shape[0]
    RD = GH // E
    I = expert_wg.shape[1]
    T = b * S

    x = hidden_states.reshape(T, H)
    wihT = gru_w_ih.T
    whhT = gru_w_hh.T
    exprWT = expr_W.T
    ctxWT = ctx_W.T
    ctxb2 = ctx_b.reshape(1, GH)
    temp2 = temp.reshape(1, 1)

    BT1 = min(256, T)
    nT1 = T // BT1
    xw, expr, hn0, _xsum = pl.pallas_call(
        functools.partial(_pre_kernel, nT=nT1, S=T),
        grid=(nT1,),
        in_specs=[
            pl.BlockSpec((BT1, H), lambda t: (t, 0)),
            pl.BlockSpec((H, threeGH), lambda t: (0, 0)),
            pl.BlockSpec((H, GH), lambda t: (0, 0)),
            pl.BlockSpec((H, GH), lambda t: (0, 0)),
            pl.BlockSpec((1, GH), lambda t: (0, 0)),
        ],
        out_specs=[
            pl.BlockSpec((BT1, threeGH), lambda t: (t, 0)),
            pl.BlockSpec((BT1, GH), lambda t: (t, 0)),
            pl.BlockSpec((1, GH), lambda t: (0, 0)),
            pl.BlockSpec((1, H), lambda t: (0, 0)),
        ],
        out_shape=[
            jax.ShapeDtypeStruct((T, threeGH), F32),
            jax.ShapeDtypeStruct((T, GH), F32),
            jax.ShapeDtypeStruct((1, GH), F32),
            jax.ShapeDtypeStruct((1, H), F32),
        ],
    )(x, wihT, exprWT, ctxWT, ctxb2)

    CG = 512
    nC = T // CG
    routing = pl.pallas_call(
        functools.partial(_gru_kernel, CG=CG, GH=GH),
        grid=(nC,),
        in_specs=[
            pl.BlockSpec((CG, threeGH), lambda c: (c, 0)),
            pl.BlockSpec((GH, threeGH), lambda c: (0, 0)),
            pl.BlockSpec((1, GH), lambda c: (0, 0)),
        ],
        out_specs=pl.BlockSpec((CG, GH), lambda c: (c, 0)),
        out_shape=jax.ShapeDtypeStruct((T, GH), F32),
        scratch_shapes=[pltpu.VMEM((1, GH), F32)],
    )(xw, whhT, hn0)

    BT3 = min(256, T)
    nT3 = T // BT3
    cs, pen = pl.pallas_call(
        functools.partial(_stats_kernel, E=E, RD=RD, GH=GH),
        grid=(nT3,),
        in_specs=[
            pl.BlockSpec((BT3, GH), lambda t: (t, 0)),
            pl.BlockSpec((BT3, GH), lambda t: (t, 0)),
        ],
        out_specs=[
            pl.BlockSpec((BT3, E), lambda t: (t, 0)),
            pl.BlockSpec((1, 1), lambda t: (0, 0)),
        ],
        out_shape=[
            jax.ShapeDtypeStruct((T, E), F32),
            jax.ShapeDtypeStruct((1, 1), F32),
        ],
    )(routing, expr)

    combine = pl.pallas_call(
        functools.partial(_topk_kernel, E=E, T=T),
        grid=(1,),
        in_specs=[
            pl.BlockSpec((T, E), lambda i: (0, 0)),
            pl.BlockSpec((1, 1), lambda i: (0, 0)),
            pl.BlockSpec((1, 1), lambda i: (0, 0)),
        ],
        out_specs=pl.BlockSpec((T, E), lambda i: (0, 0)),
        out_shape=jax.ShapeDtypeStruct((T, E), F32),
    )(cs, pen, temp2)

    BT4 = min(512, T)
    nT4 = T // BT4
    shared_out = pl.pallas_call(
        _shared_kernel,
        grid=(nT4,),
        in_specs=[
            pl.BlockSpec((BT4, H), lambda t: (t, 0)),
            pl.BlockSpec((I, H), lambda t: (0, 0)),
            pl.BlockSpec((I, H), lambda t: (0, 0)),
            pl.BlockSpec((H, I), lambda t: (0, 0)),
        ],
        out_specs=pl.BlockSpec((BT4, H), lambda t: (t, 0)),
        out_shape=jax.ShapeDtypeStruct((T, H), F32),
    )(x, shared_wg, shared_wu, shared_wd)

    out = pl.pallas_call(
        functools.partial(_experts_kernel, BT=BT4),
        grid=(E, nT4),
        in_specs=[
            pl.BlockSpec((BT4, H), lambda e, t: (t, 0)),
            pl.BlockSpec((1, I, H), lambda e, t: (e, 0, 0)),
            pl.BlockSpec((1, I, H), lambda e, t: (e, 0, 0)),
            pl.BlockSpec((1, H, I), lambda e, t: (e, 0, 0)),
            pl.BlockSpec((BT4, E), lambda e, t: (t, 0)),
            pl.BlockSpec((T, H), lambda e, t: (0, 0)),
        ],
        out_specs=pl.BlockSpec((T, H), lambda e, t: (0, 0)),
        out_shape=jax.ShapeDtypeStruct((T, H), F32),
    )(x, expert_wg, expert_wu, expert_wd, combine, shared_out)

    return out.reshape(b, S, H)
